# baseline (speedup 1.0000x reference)
import functools

import jax
import jax.numpy as jnp
from jax import lax
from jax.experimental import pallas as pl
from jax.experimental.pallas import tpu as pltpu
from jax.experimental.pallas import tpu_sc as plsc

_TB = 2


def _argmin_body(x_ref, d_ref, dn_ref, tn_ref, idx_ref):
    dfull = d_ref[...]
    dn = dn_ref[0, 0, :]
    for t in range(x_ref.shape[0]):
        xm2 = x_ref[t] * -2.0
        tn = tn_ref[0, t, :]
        dots = lax.dot_general(
            dfull, xm2, (((1,), (0,)), ((), ())),
            preferred_element_type=jnp.float32,
        )
        dist = dots + dn[:, None] + tn[None, :]
        idx_ref[0, t, :] = jnp.argmin(dist, axis=0).astype(jnp.int32)


def _distance_argmin(x3, dictionary, dict_norms, tensor_norms):
    n, c, pb = x3.shape
    d = dictionary.shape[0]
    nb = n // _TB
    idx3 = pl.pallas_call(
        _argmin_body,
        grid=(nb,),
        in_specs=[
            pl.BlockSpec((_TB, c, pb), lambda i: (i, 0, 0)),
            pl.BlockSpec((d, c), lambda i: (0, 0)),
            pl.BlockSpec((1, 1, d), lambda i: (0, 0, 0)),
            pl.BlockSpec((1, _TB, pb), lambda i: (i, 0, 0)),
        ],
        out_specs=pl.BlockSpec((1, _TB, pb), lambda i: (i, 0, 0)),
        out_shape=jax.ShapeDtypeStruct((nb, _TB, pb), jnp.int32),
    )(x3, dictionary, dict_norms.reshape(1, 1, d),
      tensor_norms.reshape(nb, _TB, pb))
    return idx3.reshape(n * pb)


def _sc_gather_nchw(table, idx, n_img, pb):
    info = plsc.get_sparse_core_info()
    nw = info.num_cores * info.num_subcores
    b = idx.shape[0]
    c = table.shape[1]
    b_per_w = b // nw
    wpi = nw // n_img
    mesh = plsc.VectorSubcoreMesh(core_axis_name="c", subcore_axis_name="s")

    @functools.partial(
        pl.kernel, mesh=mesh,
        out_type=jax.ShapeDtypeStruct((n_img, c, pb), jnp.float32),
        compiler_params=pltpu.CompilerParams(
            use_tc_tiling_on_sc=False, needs_layout_passes=False),
        scratch_types=[
            pltpu.VMEM((b_per_w,), jnp.int32),
            pltpu.VMEM((b_per_w, c), jnp.float32),
            pltpu.VMEM((c, b_per_w), jnp.float32),
            pltpu.SemaphoreType.DMA,
        ],
    )
    def gather_kernel(table_hbm, idx_hbm, out_hbm, idx_v, rows_v, rows_t, sem):
        wid = lax.axis_index("s") * info.num_cores + lax.axis_index("c")
        base = wid * b_per_w
        img = wid // wpi
        hw0 = (wid % wpi) * b_per_w
        pltpu.sync_copy(idx_hbm.at[pl.ds(base, b_per_w)], idx_v)
        pltpu.async_copy(table_hbm.at[idx_v], rows_v, sem).wait()
        lanes = lax.iota(jnp.int32, 16)
        for ch in range(c):
            col = jnp.full((16,), ch, jnp.int32)
            for i0 in range(0, b_per_w, 16):
                vals = plsc.load_gather(rows_v, [i0 + lanes, col])
                rows_t[ch, pl.ds(i0, 16)] = vals
        pltpu.sync_copy(rows_t, out_hbm.at[img, :, pl.ds(hw0, b_per_w)])

    return gather_kernel(table, idx)


def kernel(inputs, dictionary):
    n, c, h, w = inputs.shape
    x3 = inputs.reshape(n, c, h * w)
    dict_norms = jnp.sum(dictionary ** 2, axis=-1)
    tensor_norms = jnp.sum(
        jnp.transpose(inputs, (0, 2, 3, 1)) ** 2, axis=-1)
    idx_flat = _distance_argmin(
        x3, dictionary, dict_norms, tensor_norms.reshape(n, h * w))
    emb3 = _sc_gather_nchw(dictionary, idx_flat, n, h * w)
    embedded = emb3.reshape(n, c, h, w)
    idxs = idx_flat.reshape(n, h, w)
    embedded_pt = lax.stop_gradient(embedded) + (
        inputs - lax.stop_gradient(inputs))
    return (embedded, embedded_pt, idxs)


# speedup vs baseline: 1.0143x; 1.0143x over previous
import functools

import jax
import jax.numpy as jnp
from jax import lax
from jax.experimental import pallas as pl
from jax.experimental.pallas import tpu as pltpu
from jax.experimental.pallas import tpu_sc as plsc

_TB = 2


def _argmin_body(x_ref, d_ref, dn_ref, tn_ref, idx_ref):
    dfull = d_ref[...]
    dn = dn_ref[0, 0, :]
    for t in range(x_ref.shape[0]):
        xm2 = x_ref[t] * -2.0
        tn = tn_ref[0, t, :]
        dots = lax.dot_general(
            dfull, xm2, (((1,), (0,)), ((), ())),
            preferred_element_type=jnp.float32,
        )
        dist = dots + dn[:, None] + tn[None, :]
        idx_ref[0, t, :] = jnp.argmin(dist, axis=0).astype(jnp.int32)


def _distance_argmin(x3, dictionary, dict_norms, tensor_norms):
    n, c, pb = x3.shape
    d = dictionary.shape[0]
    nb = n // _TB
    idx3 = pl.pallas_call(
        _argmin_body,
        grid=(nb,),
        in_specs=[
            pl.BlockSpec((_TB, c, pb), lambda i: (i, 0, 0)),
            pl.BlockSpec((d, c), lambda i: (0, 0)),
            pl.BlockSpec((1, 1, d), lambda i: (0, 0, 0)),
            pl.BlockSpec((1, _TB, pb), lambda i: (i, 0, 0)),
        ],
        out_specs=pl.BlockSpec((1, _TB, pb), lambda i: (i, 0, 0)),
        out_shape=jax.ShapeDtypeStruct((nb, _TB, pb), jnp.int32),
    )(x3, dictionary, dict_norms.reshape(1, 1, d),
      tensor_norms.reshape(nb, _TB, pb))
    return idx3.reshape(n * pb)


def _sc_gather(table, idx):
    info = plsc.get_sparse_core_info()
    nw = info.num_cores * info.num_subcores
    b = idx.shape[0]
    d = table.shape[1]
    b_per_w = b // nw
    mesh = plsc.VectorSubcoreMesh(core_axis_name="c", subcore_axis_name="s")

    @functools.partial(
        pl.kernel, mesh=mesh,
        out_type=jax.ShapeDtypeStruct((b, d), jnp.float32),
        compiler_params=pltpu.CompilerParams(use_tc_tiling_on_sc=False),
        scratch_types=[
            pltpu.VMEM((b_per_w,), jnp.int32),
            pltpu.VMEM((b_per_w, d), jnp.float32),
            pltpu.SemaphoreType.DMA,
        ],
    )
    def gather_kernel(table_hbm, idx_hbm, out_hbm, idx_v, rows_v, sem):
        wid = lax.axis_index("s") * info.num_cores + lax.axis_index("c")
        base = wid * b_per_w
        pltpu.sync_copy(idx_hbm.at[pl.ds(base, b_per_w)], idx_v)
        pltpu.async_copy(table_hbm.at[idx_v], rows_v, sem).wait()
        pltpu.sync_copy(rows_v, out_hbm.at[pl.ds(base, b_per_w)])

    return gather_kernel(table, idx)


def kernel(inputs, dictionary):
    n, c, h, w = inputs.shape
    x3 = inputs.reshape(n, c, h * w)
    dict_norms = jnp.sum(dictionary ** 2, axis=-1)
    tensor_norms = jnp.sum(
        jnp.transpose(inputs, (0, 2, 3, 1)) ** 2, axis=-1)
    idx_flat = _distance_argmin(
        x3, dictionary, dict_norms, tensor_norms.reshape(n, h * w))
    emb_flat = _sc_gather(dictionary, idx_flat)
    embedded = jnp.transpose(emb_flat.reshape(n, h, w, c), (0, 3, 1, 2))
    idxs = idx_flat.reshape(n, h, w)
    embedded_pt = lax.stop_gradient(embedded) + (
        inputs - lax.stop_gradient(inputs))
    return (embedded, embedded_pt, idxs)
